# baseline (speedup 1.0000x reference)
import functools

import jax
import jax.numpy as jnp
from jax import lax
from jax.experimental import pallas as pl
from jax.experimental.pallas import tpu as pltpu
from jax.experimental.pallas import tpu_sc as plsc

_LOSS_WEIGHT = 0.01
_TAKE_DNUMS = lax.GatherDimensionNumbers(
    offset_dims=(), collapsed_slice_dims=(0,), start_index_map=(0,))


def _lane_bcast(x, idx):
    return lax.gather(
        x, idx[:, None], _TAKE_DNUMS, (1,),
        mode=lax.GatherScatterMode.PROMISE_IN_BOUNDS)
_L = 16
_NUM_CORES = 2
_NUM_SUBCORES = 16
_CHUNK = 1024


def _sc_body(num_tokens, num_experts, rw_hbm, out_hbm, buf, part):
    cid = lax.axis_index("c")
    sid = lax.axis_index("s")
    wid = sid * _NUM_CORES + cid
    zeros = jnp.zeros((_L,), dtype=jnp.float32)
    lane_iota = lax.broadcasted_iota(jnp.int32, (_L,), 0)
    shuffle_idx = [lane_iota ^ d for d in (1, 2, 4, 8)]
    n_groups = num_experts // _L

    def chunk_step(c, accs):
        pltpu.sync_copy(
            rw_hbm.at[wid, pl.ds(c * _CHUNK * num_experts, _CHUNK * num_experts)],
            buf)

        def token_step(t, carry):
            sums, cnts = carry
            base = t * num_experts
            v = [buf[pl.ds(base + j * _L, _L)] for j in range(n_groups)]
            a = jnp.maximum(v[0], v[1])
            b = jnp.minimum(v[0], v[1])
            c2 = jnp.maximum(v[2], v[3])
            d = jnp.minimum(v[2], v[3])
            m1 = jnp.maximum(a, c2)
            m2 = jnp.maximum(jnp.minimum(a, c2), jnp.maximum(b, d))
            for sidx in shuffle_idx:
                b1 = _lane_bcast(m1, sidx)
                b2 = _lane_bcast(m2, sidx)
                m2 = jnp.maximum(jnp.minimum(m1, b1), jnp.maximum(m2, b2))
                m1 = jnp.maximum(m1, b1)
            th = m2
            sums = tuple(sums[j] + v[j] for j in range(n_groups))
            cnts = tuple(
                cnts[j] + jnp.where(v[j] >= th, 1.0, 0.0)
                for j in range(n_groups))
            return (sums, cnts)

        return lax.fori_loop(0, _CHUNK, token_step, accs, unroll=8)

    init = ((zeros,) * n_groups, (zeros,) * n_groups)
    sums, cnts = lax.fori_loop(0, num_tokens // _CHUNK, chunk_step, init)

    acc = zeros
    for j in range(n_groups):
        acc = acc + sums[j] * cnts[j]
    part[...] = acc
    pltpu.sync_copy(part, out_hbm.at[wid])


def kernel(router_weights, n_routed_experts, num_experts_per_tok):
    num_layers, num_tokens, num_experts = router_weights.shape
    rw = router_weights.astype(jnp.float32).reshape(
        num_layers, num_tokens * num_experts)
    num_workers = _NUM_CORES * _NUM_SUBCORES
    assert num_layers == num_workers and num_experts == 4 * _L
    assert num_tokens % _CHUNK == 0

    run = pl.kernel(
        functools.partial(_sc_body, num_tokens, num_experts),
        out_type=jax.ShapeDtypeStruct((num_workers, _L), jnp.float32),
        mesh=plsc.VectorSubcoreMesh(core_axis_name="c", subcore_axis_name="s"),
        scratch_types=[
            pltpu.VMEM((_CHUNK * num_experts,), jnp.float32),
            pltpu.VMEM((_L,), jnp.float32),
        ],
        compiler_params=pltpu.CompilerParams(
            needs_layout_passes=False, disable_bounds_checks=True),
    )
    partials = run(rw)
    scale = n_routed_experts / (num_tokens * num_experts_per_tok)
    return partials.sum() * jnp.float32(scale / num_tokens * _LOSS_WEIGHT)


# speedup vs baseline: 1.2791x; 1.2791x over previous
import functools

import jax
import jax.numpy as jnp
from jax import lax
from jax.experimental import pallas as pl
from jax.experimental.pallas import tpu as pltpu
from jax.experimental.pallas import tpu_sc as plsc

_LOSS_WEIGHT = 0.01
_TAKE_DNUMS = lax.GatherDimensionNumbers(
    offset_dims=(), collapsed_slice_dims=(0,), start_index_map=(0,))


def _lane_bcast(x, idx):
    return lax.gather(
        x, idx[:, None], _TAKE_DNUMS, (1,),
        mode=lax.GatherScatterMode.PROMISE_IN_BOUNDS)
_L = 16
_NUM_CORES = 2
_NUM_SUBCORES = 16
_CHUNK = 1024


def _sc_body(num_tokens, num_experts, rw_hbm, out_hbm, buf, part):
    cid = lax.axis_index("c")
    sid = lax.axis_index("s")
    wid = sid * _NUM_CORES + cid
    zeros = jnp.zeros((_L,), dtype=jnp.float32)
    idx0 = jnp.zeros((_L,), dtype=jnp.int32)
    idx1 = jnp.ones((_L,), dtype=jnp.int32)
    n_groups = num_experts // _L

    def chunk_step(c, accs):
        pltpu.sync_copy(
            rw_hbm.at[wid, pl.ds(c * _CHUNK * num_experts, _CHUNK * num_experts)],
            buf)

        @plsc.parallel_loop(0, _CHUNK, carry=accs, unroll=8)
        def token_loop(t, carry):
            sums, cnts = carry
            base = t * num_experts
            v = [buf[pl.ds(base + j * _L, _L)] for j in range(n_groups)]
            a = jnp.maximum(v[0], v[1])
            b = jnp.minimum(v[0], v[1])
            c2 = jnp.maximum(v[2], v[3])
            d = jnp.minimum(v[2], v[3])
            m1 = jnp.maximum(a, c2)
            m2 = jnp.maximum(jnp.minimum(a, c2), jnp.maximum(b, d))
            sk, sv = plsc.sort_key_val(m1, m2, descending=True)
            t1 = _lane_bcast(sk, idx1)
            t2 = _lane_bcast(sv, idx0)
            th = jnp.maximum(t1, t2)
            sums = tuple(sums[j] + v[j] for j in range(n_groups))
            cnts = tuple(
                cnts[j] + jnp.where(v[j] >= th, 1.0, 0.0)
                for j in range(n_groups))
            return (sums, cnts)

        return token_loop

    init = ((zeros,) * n_groups, (zeros,) * n_groups)
    sums, cnts = lax.fori_loop(0, num_tokens // _CHUNK, chunk_step, init)

    acc = zeros
    for j in range(n_groups):
        acc = acc + sums[j] * cnts[j]
    part[...] = acc
    pltpu.sync_copy(part, out_hbm.at[wid])


def kernel(router_weights, n_routed_experts, num_experts_per_tok):
    num_layers, num_tokens, num_experts = router_weights.shape
    rw = router_weights.astype(jnp.float32).reshape(
        num_layers, num_tokens * num_experts)
    num_workers = _NUM_CORES * _NUM_SUBCORES
    assert num_layers == num_workers and num_experts == 4 * _L
    assert num_tokens % _CHUNK == 0

    run = pl.kernel(
        functools.partial(_sc_body, num_tokens, num_experts),
        out_type=jax.ShapeDtypeStruct((num_workers, _L), jnp.float32),
        mesh=plsc.VectorSubcoreMesh(core_axis_name="c", subcore_axis_name="s"),
        scratch_types=[
            pltpu.VMEM((_CHUNK * num_experts,), jnp.float32),
            pltpu.VMEM((_L,), jnp.float32),
        ],
        compiler_params=pltpu.CompilerParams(
            needs_layout_passes=False, disable_bounds_checks=True),
    )
    partials = run(rw)
    scale = n_routed_experts / (num_tokens * num_experts_per_tok)
    return partials.sum() * jnp.float32(scale / num_tokens * _LOSS_WEIGHT)


# speedup vs baseline: 1.5333x; 1.1987x over previous
import functools

import jax
import jax.numpy as jnp
from jax import lax
from jax.experimental import pallas as pl
from jax.experimental.pallas import tpu as pltpu
from jax.experimental.pallas import tpu_sc as plsc

_LOSS_WEIGHT = 0.01
_TAKE_DNUMS = lax.GatherDimensionNumbers(
    offset_dims=(), collapsed_slice_dims=(0,), start_index_map=(0,))


def _lane_bcast(x, idx):
    return lax.gather(
        x, idx[:, None], _TAKE_DNUMS, (1,),
        mode=lax.GatherScatterMode.PROMISE_IN_BOUNDS)
_L = 16
_NUM_CORES = 2
_NUM_SUBCORES = 16
_CHUNK = 512


def _sc_body(num_tokens, num_experts, rw_hbm, out_hbm, buf, part):
    cid = lax.axis_index("c")
    sid = lax.axis_index("s")
    wid = sid * _NUM_CORES + cid
    zeros = jnp.zeros((_L,), dtype=jnp.float32)
    idx0 = jnp.zeros((_L,), dtype=jnp.int32)
    idx1 = jnp.ones((_L,), dtype=jnp.int32)
    n_groups = num_experts // _L

    def chunk_step(c, accs):
        pltpu.sync_copy(rw_hbm.at[wid, pl.ds(c * _CHUNK, _CHUNK)], buf)

        @plsc.parallel_loop(0, _CHUNK, carry=accs, unroll=8)
        def token_loop(t, carry):
            sums, cnts = carry
            v = [buf[t, pl.ds(j * _L, _L)] for j in range(n_groups)]
            a = jnp.maximum(v[0], v[1])
            b = jnp.minimum(v[0], v[1])
            c2 = jnp.maximum(v[2], v[3])
            d = jnp.minimum(v[2], v[3])
            m1 = jnp.maximum(a, c2)
            m2 = jnp.maximum(jnp.minimum(a, c2), jnp.maximum(b, d))
            sk, sv = plsc.sort_key_val(m1, m2, descending=True)
            t1 = _lane_bcast(sk, idx1)
            t2 = _lane_bcast(sv, idx0)
            th = jnp.maximum(t1, t2)
            sums = tuple(sums[j] + v[j] for j in range(n_groups))
            cnts = tuple(
                cnts[j] + jnp.where(v[j] >= th, 1.0, 0.0)
                for j in range(n_groups))
            return (sums, cnts)

        return token_loop

    init = ((zeros,) * n_groups, (zeros,) * n_groups)
    sums, cnts = lax.fori_loop(0, num_tokens // _CHUNK, chunk_step, init)

    acc = zeros
    for j in range(n_groups):
        acc = acc + sums[j] * cnts[j]
    part[...] = acc
    pltpu.sync_copy(part, out_hbm.at[wid])


def kernel(router_weights, n_routed_experts, num_experts_per_tok):
    num_layers, num_tokens, num_experts = router_weights.shape
    rw = router_weights.astype(jnp.float32)
    num_workers = _NUM_CORES * _NUM_SUBCORES
    assert num_layers == num_workers and num_experts == 4 * _L
    assert num_tokens % _CHUNK == 0

    run = pl.kernel(
        functools.partial(_sc_body, num_tokens, num_experts),
        out_type=jax.ShapeDtypeStruct((num_workers, _L), jnp.float32),
        mesh=plsc.VectorSubcoreMesh(core_axis_name="c", subcore_axis_name="s"),
        scratch_types=[
            pltpu.VMEM((_CHUNK, num_experts), jnp.float32),
            pltpu.VMEM((_L,), jnp.float32),
        ],
        compiler_params=pltpu.CompilerParams(
            needs_layout_passes=False, disable_bounds_checks=True),
    )
    partials = run(rw)
    scale = n_routed_experts / (num_tokens * num_experts_per_tok)
    return partials.sum() * jnp.float32(scale / num_tokens * _LOSS_WEIGHT)
